# baseline (speedup 1.0000x reference)
import math

import jax
import jax.numpy as jnp
from jax.experimental import pallas as pl

B, S, D_IN, D_MODEL, H, L, KNN = 16, 128, 64, 256, 8, 2, 16
D_FF = 1024
DH = D_MODEL // H
N = B * S
ROW_BLK = 128
NUM_ROW_BLKS = N // ROW_BLK
F32 = jnp.float32


def _layernorm(x, g, b):
    m = jnp.mean(x, axis=-1, keepdims=True)
    v = jnp.mean((x - m) ** 2, axis=-1, keepdims=True)
    return (x - m) / jnp.sqrt(v + 1e-5) * g + b


def _fwd_kernel(x_ref, win_ref, bin_ref, wq_ref, bq_ref, wk_ref, bk_ref,
                wv_ref, bv_ref, wo_ref, bo_ref, w1_ref, b1_ref, w2_ref,
                b2_ref, g1_ref, be1_ref, g2_ref, be2_ref, wdown_ref,
                bdown_ref, out_ref):
    xb = x_ref[0]
    h = jnp.dot(xb, win_ref[...], preferred_element_type=F32) + bin_ref[...]
    scale = 1.0 / math.sqrt(DH)
    for l in range(L):
        q = jnp.dot(h, wq_ref[l], preferred_element_type=F32) + bq_ref[l]
        k = jnp.dot(h, wk_ref[l], preferred_element_type=F32) + bk_ref[l]
        v = jnp.dot(h, wv_ref[l], preferred_element_type=F32) + bv_ref[l]
        heads = []
        for hh in range(H):
            sl = slice(hh * DH, (hh + 1) * DH)
            qh, kh, vh = q[:, sl], k[:, sl], v[:, sl]
            s = jax.lax.dot_general(qh, kh, (((1,), (1,)), ((), ())),
                                    preferred_element_type=F32) * scale
            s = s - jnp.max(s, axis=-1, keepdims=True)
            e = jnp.exp(s)
            a = e / jnp.sum(e, axis=-1, keepdims=True)
            heads.append(jnp.dot(a, vh, preferred_element_type=F32))
        o = jnp.concatenate(heads, axis=1)
        h = _layernorm(h + jnp.dot(o, wo_ref[l], preferred_element_type=F32)
                       + bo_ref[l], g1_ref[l], be1_ref[l])
        ff = jnp.maximum(
            jnp.dot(h, w1_ref[l], preferred_element_type=F32) + b1_ref[l], 0.0)
        ff = jnp.dot(ff, w2_ref[l], preferred_element_type=F32) + b2_ref[l]
        h = _layernorm(h + ff, g2_ref[l], be2_ref[l])
    out_ref[0] = (jnp.dot(h, wdown_ref[...], preferred_element_type=F32)
                  + bdown_ref[...])


def _knn_kernel(xrec_ref, loss_ref):
    i = pl.program_id(0)
    xr = xrec_ref[...]
    rows = xrec_ref[pl.ds(i * ROW_BLK, ROW_BLK), :]
    sq_all = jnp.sum(xr * xr, axis=1)[None, :]
    sq_rows = jnp.sum(rows * rows, axis=1)[:, None]
    prod = jax.lax.dot_general(rows, xr, (((1,), (1,)), ((), ())),
                               preferred_element_type=F32)
    d = sq_rows + sq_all - 2.0 * prod
    col = jax.lax.broadcasted_iota(jnp.int32, d.shape, 1)
    total = jnp.zeros((), F32)
    for t in range(KNN):
        m = jnp.min(d, axis=1, keepdims=True)
        total = total + jnp.sum(m)
        if t + 1 < KNN:
            am = jnp.argmin(d, axis=1)[:, None]
            d = jnp.where(col == am, jnp.inf, d)

    @pl.when(i == 0)
    def _():
        loss_ref[...] = jnp.zeros((1, 1), F32)

    loss_ref[...] += jnp.reshape(total, (1, 1))


def kernel(x, W_in, b_in, Wq, bq, Wk, bk, Wv, bv, Wo, bo, W1, b1, W2, b2,
           g1, be1, g2, be2, W_down, b_down):
    const2 = lambda b: (0, 0)
    const3 = lambda b: (0, 0, 0)
    full2 = lambda a: pl.BlockSpec(a.shape, const2)

    in_specs = [
        pl.BlockSpec((1, S, D_IN), lambda b: (b, 0, 0)),
        full2(W_in),
        pl.BlockSpec((1, D_MODEL), const2),
        pl.BlockSpec((L, D_MODEL, D_MODEL), const3),
        pl.BlockSpec((L, 1, D_MODEL), const3),
        pl.BlockSpec((L, D_MODEL, D_MODEL), const3),
        pl.BlockSpec((L, 1, D_MODEL), const3),
        pl.BlockSpec((L, D_MODEL, D_MODEL), const3),
        pl.BlockSpec((L, 1, D_MODEL), const3),
        pl.BlockSpec((L, D_MODEL, D_MODEL), const3),
        pl.BlockSpec((L, 1, D_MODEL), const3),
        pl.BlockSpec((L, D_MODEL, D_FF), const3),
        pl.BlockSpec((L, 1, D_FF), const3),
        pl.BlockSpec((L, D_FF, D_MODEL), const3),
        pl.BlockSpec((L, 1, D_MODEL), const3),
        pl.BlockSpec((L, 1, D_MODEL), const3),
        pl.BlockSpec((L, 1, D_MODEL), const3),
        pl.BlockSpec((L, 1, D_MODEL), const3),
        pl.BlockSpec((L, 1, D_MODEL), const3),
        full2(W_down),
        pl.BlockSpec((1, D_IN), const2),
    ]

    x_rec_b = pl.pallas_call(
        _fwd_kernel,
        grid=(B,),
        in_specs=in_specs,
        out_specs=pl.BlockSpec((1, S, D_IN), lambda b: (b, 0, 0)),
        out_shape=jax.ShapeDtypeStruct((B, S, D_IN), F32),
    )(x, W_in, b_in[None, :], Wq, bq[:, None, :], Wk, bk[:, None, :],
      Wv, bv[:, None, :], Wo, bo[:, None, :], W1, b1[:, None, :],
      W2, b2[:, None, :], g1[:, None, :], be1[:, None, :],
      g2[:, None, :], be2[:, None, :], W_down, b_down[None, :])

    x_rec = x_rec_b.reshape(N, D_IN)

    loss = pl.pallas_call(
        _knn_kernel,
        grid=(NUM_ROW_BLKS,),
        in_specs=[pl.BlockSpec((N, D_IN), lambda i: (0, 0))],
        out_specs=pl.BlockSpec((1, 1), lambda i: (0, 0)),
        out_shape=jax.ShapeDtypeStruct((1, 1), F32),
    )(x_rec)

    return loss[0, 0], x_rec


# speedup vs baseline: 2.1935x; 2.1935x over previous
import math

import jax
import jax.numpy as jnp
from jax.experimental import pallas as pl
from jax.experimental.pallas import tpu as pltpu

B, S, D_IN, D_MODEL, H, L, KNN = 16, 128, 64, 256, 8, 2, 16
D_FF = 1024
DH = D_MODEL // H
N = B * S
ROW_BLK = 128
NUM_ROW_BLKS = N // ROW_BLK
F32 = jnp.float32


def _layernorm(x, g, b):
    m = jnp.mean(x, axis=-1, keepdims=True)
    v = jnp.mean((x - m) ** 2, axis=-1, keepdims=True)
    return (x - m) / jnp.sqrt(v + 1e-5) * g + b


def _fwd_kernel(x_ref, win_ref, bin_ref, wq_ref, bq_ref, wk_ref, bk_ref,
                wv_ref, bv_ref, wo_ref, bo_ref, w1_ref, b1_ref, w2_ref,
                b2_ref, g1_ref, be1_ref, g2_ref, be2_ref, wdown_ref,
                bdown_ref, out_ref):
    xb = x_ref[0]
    h = jnp.dot(xb, win_ref[...], preferred_element_type=F32) + bin_ref[...]
    scale = 1.0 / math.sqrt(DH)
    for l in range(L):
        q = jnp.dot(h, wq_ref[l], preferred_element_type=F32) + bq_ref[l]
        k = jnp.dot(h, wk_ref[l], preferred_element_type=F32) + bk_ref[l]
        v = jnp.dot(h, wv_ref[l], preferred_element_type=F32) + bv_ref[l]
        heads = []
        for hh in range(H):
            sl = slice(hh * DH, (hh + 1) * DH)
            qh, kh, vh = q[:, sl], k[:, sl], v[:, sl]
            s = jax.lax.dot_general(qh, kh, (((1,), (1,)), ((), ())),
                                    preferred_element_type=F32) * scale
            s = s - jnp.max(s, axis=-1, keepdims=True)
            e = jnp.exp(s)
            a = e / jnp.sum(e, axis=-1, keepdims=True)
            heads.append(jnp.dot(a, vh, preferred_element_type=F32))
        o = jnp.concatenate(heads, axis=1)
        h = _layernorm(h + jnp.dot(o, wo_ref[l], preferred_element_type=F32)
                       + bo_ref[l], g1_ref[l], be1_ref[l])
        ff = jnp.maximum(
            jnp.dot(h, w1_ref[l], preferred_element_type=F32) + b1_ref[l], 0.0)
        ff = jnp.dot(ff, w2_ref[l], preferred_element_type=F32) + b2_ref[l]
        h = _layernorm(h + ff, g2_ref[l], be2_ref[l])
    out_ref[0] = (jnp.dot(h, wdown_ref[...], preferred_element_type=F32)
                  + bdown_ref[...])


def _knn_kernel(xrec_ref, loss_ref):
    i = pl.program_id(0)
    xr = xrec_ref[...]
    rows = xrec_ref[pl.ds(i * ROW_BLK, ROW_BLK), :]
    sq_all = jnp.sum(xr * xr, axis=1)[None, :]
    sq_rows = jnp.sum(rows * rows, axis=1)[:, None]
    prod = jax.lax.dot_general(rows, xr, (((1,), (1,)), ((), ())),
                               preferred_element_type=F32)
    d = sq_rows + sq_all - 2.0 * prod
    total = jnp.zeros((), F32)
    for t in range(KNN):
        m = jnp.min(d, axis=1, keepdims=True)
        total = total + jnp.sum(m)
        if t + 1 < KNN:
            d = jnp.where(d == m, jnp.inf, d)
    loss_ref[...] = jnp.reshape(total, (1, 1, 1))


def kernel(x, W_in, b_in, Wq, bq, Wk, bk, Wv, bv, Wo, bo, W1, b1, W2, b2,
           g1, be1, g2, be2, W_down, b_down):
    const2 = lambda b: (0, 0)
    const3 = lambda b: (0, 0, 0)
    full2 = lambda a: pl.BlockSpec(a.shape, const2)

    in_specs = [
        pl.BlockSpec((1, S, D_IN), lambda b: (b, 0, 0)),
        full2(W_in),
        pl.BlockSpec((1, D_MODEL), const2),
        pl.BlockSpec((L, D_MODEL, D_MODEL), const3),
        pl.BlockSpec((L, 1, D_MODEL), const3),
        pl.BlockSpec((L, D_MODEL, D_MODEL), const3),
        pl.BlockSpec((L, 1, D_MODEL), const3),
        pl.BlockSpec((L, D_MODEL, D_MODEL), const3),
        pl.BlockSpec((L, 1, D_MODEL), const3),
        pl.BlockSpec((L, D_MODEL, D_MODEL), const3),
        pl.BlockSpec((L, 1, D_MODEL), const3),
        pl.BlockSpec((L, D_MODEL, D_FF), const3),
        pl.BlockSpec((L, 1, D_FF), const3),
        pl.BlockSpec((L, D_FF, D_MODEL), const3),
        pl.BlockSpec((L, 1, D_MODEL), const3),
        pl.BlockSpec((L, 1, D_MODEL), const3),
        pl.BlockSpec((L, 1, D_MODEL), const3),
        pl.BlockSpec((L, 1, D_MODEL), const3),
        pl.BlockSpec((L, 1, D_MODEL), const3),
        full2(W_down),
        pl.BlockSpec((1, D_IN), const2),
    ]

    x_rec_b = pl.pallas_call(
        _fwd_kernel,
        grid=(B,),
        in_specs=in_specs,
        out_specs=pl.BlockSpec((1, S, D_IN), lambda b: (b, 0, 0)),
        out_shape=jax.ShapeDtypeStruct((B, S, D_IN), F32),
        compiler_params=pltpu.CompilerParams(
            dimension_semantics=("parallel",)),
    )(x, W_in, b_in[None, :], Wq, bq[:, None, :], Wk, bk[:, None, :],
      Wv, bv[:, None, :], Wo, bo[:, None, :], W1, b1[:, None, :],
      W2, b2[:, None, :], g1[:, None, :], be1[:, None, :],
      g2[:, None, :], be2[:, None, :], W_down, b_down[None, :])

    x_rec = x_rec_b.reshape(N, D_IN)

    partial = pl.pallas_call(
        _knn_kernel,
        grid=(NUM_ROW_BLKS,),
        in_specs=[pl.BlockSpec((N, D_IN), lambda i: (0, 0))],
        out_specs=pl.BlockSpec((1, 1, 1), lambda i: (i, 0, 0)),
        out_shape=jax.ShapeDtypeStruct((NUM_ROW_BLKS, 1, 1), F32),
        compiler_params=pltpu.CompilerParams(
            dimension_semantics=("parallel",)),
    )(x_rec)

    return jnp.sum(partial), x_rec


# speedup vs baseline: 3.3255x; 1.5161x over previous
import math

import jax
import jax.numpy as jnp
from jax.experimental import pallas as pl
from jax.experimental.pallas import tpu as pltpu

B, S, D_IN, D_MODEL, H, L, KNN = 16, 128, 64, 256, 8, 2, 16
D_FF = 1024
DH = D_MODEL // H
N = B * S
ROW_BLK = 128
NUM_ROW_BLKS = N // ROW_BLK
F32 = jnp.float32


def _layernorm(x, g, b):
    m = jnp.mean(x, axis=-1, keepdims=True)
    v = jnp.mean((x - m) ** 2, axis=-1, keepdims=True)
    return (x - m) / jnp.sqrt(v + 1e-5) * g + b


NB = 4


def _fwd_kernel(x_ref, win_ref, bin_ref, wqkv_ref, bqkv_ref, wo_ref, bo_ref,
                w1_ref, b1_ref, w2_ref, b2_ref, g1_ref, be1_ref, g2_ref,
                be2_ref, wdown_ref, bdown_ref, out_ref):
    M = NB * S
    xb = x_ref[...].reshape(M, D_IN)
    h = jnp.dot(xb, win_ref[...], preferred_element_type=F32) + bin_ref[...]
    for l in range(L):
        qkv = (jnp.dot(h, wqkv_ref[l], preferred_element_type=F32)
               + bqkv_ref[l])
        o_rows = []
        for b in range(NB):
            rs = slice(b * S, (b + 1) * S)
            heads = []
            for hh in range(H):
                qs = slice(hh * DH, (hh + 1) * DH)
                ks = slice(D_MODEL + hh * DH, D_MODEL + (hh + 1) * DH)
                vs = slice(2 * D_MODEL + hh * DH, 2 * D_MODEL + (hh + 1) * DH)
                qh, kh, vh = qkv[rs, qs], qkv[rs, ks], qkv[rs, vs]
                s = jax.lax.dot_general(qh, kh, (((1,), (1,)), ((), ())),
                                        preferred_element_type=F32)
                e = jnp.exp(s)
                r = 1.0 / jnp.sum(e, axis=-1, keepdims=True)
                heads.append(
                    jnp.dot(e, vh, preferred_element_type=F32) * r)
            o_rows.append(jnp.concatenate(heads, axis=1))
        o = jnp.concatenate(o_rows, axis=0)
        h = _layernorm(h + jnp.dot(o, wo_ref[l], preferred_element_type=F32)
                       + bo_ref[l], g1_ref[l], be1_ref[l])
        ff = jnp.maximum(
            jnp.dot(h, w1_ref[l], preferred_element_type=F32) + b1_ref[l], 0.0)
        ff = jnp.dot(ff, w2_ref[l], preferred_element_type=F32) + b2_ref[l]
        h = _layernorm(h + ff, g2_ref[l], be2_ref[l])
    out_ref[...] = (jnp.dot(h, wdown_ref[...], preferred_element_type=F32)
                    + bdown_ref[...]).reshape(NB, S, D_IN)


def _knn_kernel(xrec_ref, loss_ref):
    i = pl.program_id(0)
    xr = xrec_ref[...]
    rows = xrec_ref[pl.ds(i * ROW_BLK, ROW_BLK), :]
    sq_all = jnp.sum(xr * xr, axis=1)[None, :]
    sq_rows = jnp.sum(rows * rows, axis=1)[:, None]
    prod = jax.lax.dot_general(rows, xr, (((1,), (1,)), ((), ())),
                               preferred_element_type=F32)
    d = sq_rows + sq_all - 2.0 * prod
    total = jnp.zeros((), F32)
    for t in range(KNN):
        m = jnp.min(d, axis=1, keepdims=True)
        total = total + jnp.sum(m)
        if t + 1 < KNN:
            d = jnp.where(d == m, jnp.inf, d)
    loss_ref[...] = jnp.reshape(total, (1, 1, 1))


def kernel(x, W_in, b_in, Wq, bq, Wk, bk, Wv, bv, Wo, bo, W1, b1, W2, b2,
           g1, be1, g2, be2, W_down, b_down):
    scale = 1.0 / math.sqrt(DH)
    Wqkv = jnp.concatenate([Wq * scale, Wk, Wv], axis=2)
    bqkv = jnp.concatenate([bq * scale, bk, bv],
                           axis=1)[:, None, :]

    const2 = lambda b: (0, 0)
    const3 = lambda b: (0, 0, 0)
    full2 = lambda a: pl.BlockSpec(a.shape, const2)

    in_specs = [
        pl.BlockSpec((NB, S, D_IN), lambda b: (b, 0, 0)),
        full2(W_in),
        pl.BlockSpec((1, D_MODEL), const2),
        pl.BlockSpec((L, D_MODEL, 3 * D_MODEL), const3),
        pl.BlockSpec((L, 1, 3 * D_MODEL), const3),
        pl.BlockSpec((L, D_MODEL, D_MODEL), const3),
        pl.BlockSpec((L, 1, D_MODEL), const3),
        pl.BlockSpec((L, D_MODEL, D_FF), const3),
        pl.BlockSpec((L, 1, D_FF), const3),
        pl.BlockSpec((L, D_FF, D_MODEL), const3),
        pl.BlockSpec((L, 1, D_MODEL), const3),
        pl.BlockSpec((L, 1, D_MODEL), const3),
        pl.BlockSpec((L, 1, D_MODEL), const3),
        pl.BlockSpec((L, 1, D_MODEL), const3),
        pl.BlockSpec((L, 1, D_MODEL), const3),
        full2(W_down),
        pl.BlockSpec((1, D_IN), const2),
    ]

    x_rec_b = pl.pallas_call(
        _fwd_kernel,
        grid=(B // NB,),
        in_specs=in_specs,
        out_specs=pl.BlockSpec((NB, S, D_IN), lambda b: (b, 0, 0)),
        out_shape=jax.ShapeDtypeStruct((B, S, D_IN), F32),
        compiler_params=pltpu.CompilerParams(
            dimension_semantics=("parallel",)),
    )(x, W_in, b_in[None, :], Wqkv, bqkv, Wo, bo[:, None, :],
      W1, b1[:, None, :], W2, b2[:, None, :], g1[:, None, :],
      be1[:, None, :], g2[:, None, :], be2[:, None, :], W_down,
      b_down[None, :])

    x_rec = x_rec_b.reshape(N, D_IN)

    partial = pl.pallas_call(
        _knn_kernel,
        grid=(NUM_ROW_BLKS,),
        in_specs=[pl.BlockSpec((N, D_IN), lambda i: (0, 0))],
        out_specs=pl.BlockSpec((1, 1, 1), lambda i: (i, 0, 0)),
        out_shape=jax.ShapeDtypeStruct((NUM_ROW_BLKS, 1, 1), F32),
        compiler_params=pltpu.CompilerParams(
            dimension_semantics=("parallel",)),
    )(x_rec)

    return jnp.sum(partial), x_rec


# speedup vs baseline: 3.4048x; 1.0238x over previous
import math

import jax
import jax.numpy as jnp
from jax.experimental import pallas as pl
from jax.experimental.pallas import tpu as pltpu

B, S, D_IN, D_MODEL, H, L, KNN = 16, 128, 64, 256, 8, 2, 16
D_FF = 1024
DH = D_MODEL // H
N = B * S
ROW_BLK = 128
NUM_ROW_BLKS = N // ROW_BLK
F32 = jnp.float32


def _layernorm(x, g, b):
    m = jnp.mean(x, axis=-1, keepdims=True)
    v = jnp.mean((x - m) ** 2, axis=-1, keepdims=True)
    return (x - m) * jax.lax.rsqrt(v + 1e-5) * g + b


NB = 4


N_FWD_STEPS = B // NB


def _fused_kernel(x_ref, win_ref, bin_ref, wqkv_ref, bqkv_ref, wo_ref, bo_ref,
                  w1_ref, b1_ref, w2_ref, b2_ref, g1_ref, be1_ref, g2_ref,
                  be2_ref, wdown_ref, bdown_ref, out_ref, loss_ref,
                  xrec_scratch):
    i = pl.program_id(0)

    @pl.when(i < N_FWD_STEPS)
    def _():
        _fwd_step(i, x_ref, win_ref, bin_ref, wqkv_ref, bqkv_ref, wo_ref,
                  bo_ref, w1_ref, b1_ref, w2_ref, b2_ref, g1_ref, be1_ref,
                  g2_ref, be2_ref, wdown_ref, bdown_ref, out_ref,
                  xrec_scratch)

    @pl.when(i >= N_FWD_STEPS)
    def _():
        _knn_step(i - N_FWD_STEPS, xrec_scratch, loss_ref)


def _fwd_step(i, x_ref, win_ref, bin_ref, wqkv_ref, bqkv_ref, wo_ref, bo_ref,
              w1_ref, b1_ref, w2_ref, b2_ref, g1_ref, be1_ref, g2_ref,
              be2_ref, wdown_ref, bdown_ref, out_ref, xrec_scratch):
    M = NB * S
    xb = x_ref[...].reshape(M, D_IN)
    h = jnp.dot(xb, win_ref[...], preferred_element_type=F32) + bin_ref[...]
    sel = (jax.lax.broadcasted_iota(jnp.int32, (H * S, H), 0) // S
           == jax.lax.broadcasted_iota(jnp.int32, (H * S, H), 1)).astype(F32)
    rep = (jax.lax.broadcasted_iota(jnp.int32, (H, D_MODEL), 0)
           == jax.lax.broadcasted_iota(jnp.int32, (H, D_MODEL), 1) // DH
           ).astype(F32)
    for l in range(L):
        qkv = (jnp.dot(h, wqkv_ref[l], preferred_element_type=F32)
               + bqkv_ref[l])
        o_rows = []
        for b in range(NB):
            rs = slice(b * S, (b + 1) * S)
            es, avs = [], []
            for hh in range(H):
                qs = slice(hh * DH, (hh + 1) * DH)
                ks = slice(D_MODEL + hh * DH, D_MODEL + (hh + 1) * DH)
                vs = slice(2 * D_MODEL + hh * DH, 2 * D_MODEL + (hh + 1) * DH)
                qh, kh, vh = qkv[rs, qs], qkv[rs, ks], qkv[rs, vs]
                s = jax.lax.dot_general(qh, kh, (((1,), (1,)), ((), ())),
                                        preferred_element_type=F32)
                e = jnp.exp(s)
                es.append(e)
                avs.append(jnp.dot(e, vh, preferred_element_type=F32))
            e_cat = jnp.concatenate(es, axis=1)
            sums = jnp.dot(e_cat, sel, preferred_element_type=F32)
            r_rep = jnp.dot(1.0 / sums, rep,
                            preferred_element_type=F32)
            o_rows.append(jnp.concatenate(avs, axis=1) * r_rep)
        o = jnp.concatenate(o_rows, axis=0)
        h = _layernorm(h + jnp.dot(o, wo_ref[l], preferred_element_type=F32)
                       + bo_ref[l], g1_ref[l], be1_ref[l])
        ff = jnp.maximum(
            jnp.dot(h, w1_ref[l], preferred_element_type=F32) + b1_ref[l], 0.0)
        ff = jnp.dot(ff, w2_ref[l], preferred_element_type=F32) + b2_ref[l]
        h = _layernorm(h + ff, g2_ref[l], be2_ref[l])
    xr = (jnp.dot(h, wdown_ref[...], preferred_element_type=F32)
          + bdown_ref[...])
    out_ref[...] = xr.reshape(NB, S, D_IN)
    xrec_scratch[pl.ds(i * M, M), :] = xr


def _knn_step(j, xrec_scratch, loss_ref):
    xr = xrec_scratch[...]
    rows = xrec_scratch[pl.ds(j * ROW_BLK, ROW_BLK), :]
    sq_all = jnp.sum(xr * xr, axis=1)[None, :]
    sq_rows = jnp.sum(rows * rows, axis=1)[:, None]
    prod = jax.lax.dot_general(rows, xr, (((1,), (1,)), ((), ())),
                               preferred_element_type=F32)
    d = sq_rows + sq_all - 2.0 * prod
    total = jnp.zeros((), F32)
    for t in range(KNN):
        m = jnp.min(d, axis=1, keepdims=True)
        total = total + jnp.sum(m)
        if t + 1 < KNN:
            d = jnp.where(d == m, jnp.inf, d)

    @pl.when(j == 0)
    def _():
        loss_ref[...] = jnp.zeros((1, 1), F32)

    loss_ref[...] += jnp.reshape(total, (1, 1))


def kernel(x, W_in, b_in, Wq, bq, Wk, bk, Wv, bv, Wo, bo, W1, b1, W2, b2,
           g1, be1, g2, be2, W_down, b_down):
    scale = 1.0 / math.sqrt(DH)
    Wqkv = jnp.concatenate([Wq * scale, Wk, Wv], axis=2)
    bqkv = jnp.concatenate([bq * scale, bk, bv],
                           axis=1)[:, None, :]

    const2 = lambda b: (0, 0)
    const3 = lambda b: (0, 0, 0)
    full2 = lambda a: pl.BlockSpec(a.shape, const2)
    clamp = lambda b: jnp.minimum(b, N_FWD_STEPS - 1)

    in_specs = [
        pl.BlockSpec((NB, S, D_IN), lambda b: (clamp(b), 0, 0)),
        full2(W_in),
        pl.BlockSpec((1, D_MODEL), const2),
        pl.BlockSpec((L, D_MODEL, 3 * D_MODEL), const3),
        pl.BlockSpec((L, 1, 3 * D_MODEL), const3),
        pl.BlockSpec((L, D_MODEL, D_MODEL), const3),
        pl.BlockSpec((L, 1, D_MODEL), const3),
        pl.BlockSpec((L, D_MODEL, D_FF), const3),
        pl.BlockSpec((L, 1, D_FF), const3),
        pl.BlockSpec((L, D_FF, D_MODEL), const3),
        pl.BlockSpec((L, 1, D_MODEL), const3),
        pl.BlockSpec((L, 1, D_MODEL), const3),
        pl.BlockSpec((L, 1, D_MODEL), const3),
        pl.BlockSpec((L, 1, D_MODEL), const3),
        pl.BlockSpec((L, 1, D_MODEL), const3),
        full2(W_down),
        pl.BlockSpec((1, D_IN), const2),
    ]

    x_rec_b, loss = pl.pallas_call(
        _fused_kernel,
        grid=(N_FWD_STEPS + NUM_ROW_BLKS,),
        in_specs=in_specs,
        out_specs=[
            pl.BlockSpec((NB, S, D_IN), lambda b: (clamp(b), 0, 0)),
            pl.BlockSpec((1, 1), lambda b: (0, 0)),
        ],
        out_shape=[
            jax.ShapeDtypeStruct((B, S, D_IN), F32),
            jax.ShapeDtypeStruct((1, 1), F32),
        ],
        scratch_shapes=[pltpu.VMEM((N, D_IN), F32)],
    )(x, W_in, b_in[None, :], Wqkv, bqkv, Wo, bo[:, None, :],
      W1, b1[:, None, :], W2, b2[:, None, :], g1[:, None, :],
      be1[:, None, :], g2[:, None, :], be2[:, None, :], W_down,
      b_down[None, :])

    return loss[0, 0], x_rec_b.reshape(N, D_IN)
